# baseline (device time: 15643 ns/iter reference)
import jax
import jax.numpy as jnp
from jax import lax
from jax.experimental import pallas as pl
from jax.experimental.pallas import tpu as pltpu

N_DEV = 4


def kernel(x, w_mat):
    m_per, k = x.shape
    n = w_mat.shape[1]
    n_per = n // N_DEV

    def body(x_ref, w_ref, out_ref, send_buf, send_sems, recv_sems):
        my = lax.axis_index("i")

        barrier_sem = pltpu.get_barrier_semaphore()
        for j in range(N_DEV):
            @pl.when(j != my)
            def _():
                pl.semaphore_signal(
                    barrier_sem, inc=1,
                    device_id=(j,), device_id_type=pl.DeviceIdType.MESH,
                )
        pl.semaphore_wait(barrier_sem, N_DEV - 1)

        for j in range(N_DEV):
            blk = jnp.maximum(
                jnp.dot(
                    x_ref[:, :],
                    w_ref[:, j * n_per:(j + 1) * n_per],
                    preferred_element_type=jnp.float32,
                ),
                0.0,
            )

            @pl.when(j == my)
            def _():
                out_ref[pl.ds(j * m_per, m_per), :] = blk

            @pl.when(j != my)
            def _():
                send_buf[j, :, :] = blk
                rdma = pltpu.make_async_remote_copy(
                    src_ref=send_buf.at[j],
                    dst_ref=out_ref.at[pl.ds(my * m_per, m_per)],
                    send_sem=send_sems.at[j],
                    recv_sem=recv_sems.at[my],
                    device_id=(j,),
                    device_id_type=pl.DeviceIdType.MESH,
                )
                rdma.start()

        for j in range(N_DEV):
            @pl.when(j != my)
            def _():
                desc = pltpu.make_async_remote_copy(
                    src_ref=send_buf.at[j],
                    dst_ref=out_ref.at[pl.ds(j * m_per, m_per)],
                    send_sem=send_sems.at[j],
                    recv_sem=recv_sems.at[j],
                    device_id=(j,),
                    device_id_type=pl.DeviceIdType.MESH,
                )
                desc.wait_recv()
                desc.wait_send()

    return pl.pallas_call(
        body,
        out_shape=jax.ShapeDtypeStruct((N_DEV * m_per, n_per), jnp.float32),
        in_specs=[
            pl.BlockSpec(memory_space=pltpu.VMEM),
            pl.BlockSpec(memory_space=pltpu.VMEM),
        ],
        out_specs=pl.BlockSpec(memory_space=pltpu.VMEM),
        scratch_shapes=[
            pltpu.VMEM((N_DEV, m_per, n_per), jnp.float32),
            pltpu.SemaphoreType.DMA((N_DEV,)),
            pltpu.SemaphoreType.DMA((N_DEV,)),
        ],
        compiler_params=pltpu.CompilerParams(collective_id=0),
    )(x, w_mat)


# device time: 5376 ns/iter; 2.9098x vs baseline; 2.9098x over previous
import jax
import jax.numpy as jnp
from jax import lax
from jax.experimental import pallas as pl
from jax.experimental.pallas import tpu as pltpu

N_DEV = 4


def kernel(x, w_mat):
    m_per, k = x.shape
    n = w_mat.shape[1]
    n_per = n // N_DEV

    def body(x_ref, w_ref, out_ref, send_buf, send_sems, recv_sems):
        my = lax.axis_index("i")

        barrier_sem = pltpu.get_barrier_semaphore()
        for j in range(N_DEV):
            @pl.when(j != my)
            def _():
                pl.semaphore_signal(
                    barrier_sem, inc=1,
                    device_id=(j,), device_id_type=pl.DeviceIdType.MESH,
                )
        pl.semaphore_wait(barrier_sem, N_DEV - 1)

        for j in range(N_DEV):
            blk = jnp.maximum(
                jnp.dot(
                    x_ref[:, :],
                    w_ref[:, j * n_per:(j + 1) * n_per],
                    preferred_element_type=jnp.float32,
                ),
                0.0,
            )

            @pl.when(j == my)
            def _():
                out_ref[pl.ds(j * m_per, m_per), :] = blk

            @pl.when(j != my)
            def _():
                send_buf[j, :, :] = blk
                rdma = pltpu.make_async_remote_copy(
                    src_ref=send_buf.at[j],
                    dst_ref=out_ref.at[pl.ds(my * m_per, m_per)],
                    send_sem=send_sems.at[j],
                    recv_sem=recv_sems.at[my],
                    device_id=(j,),
                    device_id_type=pl.DeviceIdType.MESH,
                )
                rdma.start()

        for j in range(N_DEV):
            @pl.when(j != my)
            def _():
                desc = pltpu.make_async_remote_copy(
                    src_ref=send_buf.at[j],
                    dst_ref=out_ref.at[pl.ds(j * m_per, m_per)],
                    send_sem=send_sems.at[j],
                    recv_sem=recv_sems.at[j],
                    device_id=(j,),
                    device_id_type=pl.DeviceIdType.MESH,
                )
                desc.wait_recv()
                desc.wait_send()

    return pl.pallas_call(
        body,
        out_shape=jax.ShapeDtypeStruct((N_DEV * m_per, n_per), jnp.float32),
        in_specs=[
            pl.BlockSpec(memory_space=pltpu.VMEM),
            pl.BlockSpec(memory_space=pltpu.VMEM),
        ],
        out_specs=pl.BlockSpec(memory_space=pltpu.VMEM),
        scratch_shapes=[
            pltpu.VMEM((N_DEV, m_per, n_per), jnp.float32),
            pltpu.SemaphoreType.DMA((N_DEV,)),
            pltpu.SemaphoreType.DMA((N_DEV,)),
        ],
        compiler_params=pltpu.CompilerParams(collective_id=0),
    )(x, w_mat)


import os as _os

_variant = _os.environ.get("KERNEL_VARIANT")
if _variant:
    import exp_kernels as _ek

    kernel = getattr(_ek, f"kernel_{_variant}")
